# Initial kernel scaffold; baseline (speedup 1.0000x reference)
import jax
import jax.numpy as jnp
from jax.experimental import pallas as pl


def kernel(bag_embedding, relation, head, tail, rel_embs, cell, fusion_W, fusion_b, gi1_w, gi1_b, go1_w, go1_b, gi2_w, gi2_b, go2_w, go2_b, gi3_w, gi3_b, go3_w, go3_b, parent_top, parent_mid):
    raise NotImplementedError("write your pallas kernel here")


# speedup vs baseline: 1.6584x; 1.6584x over previous
import functools

import jax
import jax.numpy as jnp
from jax import lax
from jax.experimental import pallas as pl

SENT = 768
ENT = 256
HIE = 512
N_LEAF = 53
N_MID = 36
N_TOP = 6
NREL = N_LEAF + N_MID + N_TOP
NCELL = NREL + 1

BB = 512


def _dot(a, b):
    return jnp.dot(a, b, preferred_element_type=jnp.float32)


def _dot_t(a, b):
    return lax.dot_general(a, b, (((1,), (1,)), ((), ())),
                           preferred_element_type=jnp.float32)


def _softmax(x):
    m = jnp.max(x, axis=-1, keepdims=True)
    e = jnp.exp(x - m)
    return e / jnp.sum(e, axis=-1, keepdims=True)


def _argmax(x, n):
    m = jnp.max(x, axis=-1, keepdims=True)
    io = lax.broadcasted_iota(jnp.int32, x.shape, 1)
    return jnp.min(jnp.where(x == m, io, n), axis=-1, keepdims=True)


def _znorm(z):
    return jnp.maximum(jnp.sqrt(jnp.sum(z * z, axis=-1, keepdims=True)), 1e-8)


def _take_cols(src, sel_ref, n_src, n_out, bb):
    acc = jnp.zeros((bb, n_out), dtype=jnp.float32)
    sel = sel_ref[...]
    for t in range(n_src):
        acc = acc + jnp.where(sel == t, src[:, t:t + 1], 0.0)
    return acc


def _split3(x):
    hi = x.astype(jnp.bfloat16).astype(jnp.float32)
    r = x - hi
    md = r.astype(jnp.bfloat16).astype(jnp.float32)
    return hi, md, r - md


def _dot_exact(a3, b3):
    ahi, amd, alo = a3
    bhi, bmd, blo = b3
    return (((((_dot(ahi, bhi) + _dot(ahi, bmd)) + _dot(amd, bhi))
              + _dot(amd, bmd)) + _dot(ahi, blo)) + _dot(alo, bhi))


def _hme_block(bag_ref, head_ref, tail_ref, rel_ref,
               rl_ref, rm_ref, rt_ref,
               cell_ref, chi_ref, cmd_ref, clo_ref,
               wt_ref, wb_ref, fb_ref, gb_ref, gc_ref, gbias_ref,
               rnl_ref, rnm_ref, rnt_ref, pt_ref, pm_ref,
               probs_ref, pred_ref):
    bag = bag_ref[...]
    ent = head_ref[...] - tail_ref[...]
    ne = _dot(bag, wt_ref[...]) + _dot(ent, wb_ref[...]) + fb_ref[...]

    bagg = _dot(bag, gb_ref[...]) + gbias_ref[...]
    gc = gc_ref[...]

    iota_cell = lax.broadcasted_iota(jnp.int32, (1, NCELL), 1)

    def gather_cell3(idx_col):
        oh = (idx_col == iota_cell).astype(jnp.float32)
        return _dot(oh, chi_ref[...]), _dot(oh, cmd_ref[...]), _dot(oh, clo_ref[...])

    def gate_c(c, col):
        return jax.nn.sigmoid(bagg[:, col:col + 1] + _dot(c, gc[:, col:col + 1]))

    m_top = jnp.broadcast_to(cell_ref[NCELL - 1:NCELL, :], (BB, HIE))
    i1 = jax.nn.sigmoid(bagg[:, 0:1] + _dot(m_top, gc[:, 0:1]))
    nc1 = i1 * ne + (1.0 - i1) * m_top
    o1 = jax.nn.sigmoid(bagg[:, 1:2] + _dot(nc1, gc[:, 1:2]))
    z1 = o1 * m_top + (1.0 - o1) * ne
    prob_top = _dot_t(z1, rt_ref[...]) / (_znorm(z1) * rnt_ref[...])

    m2_3 = gather_cell3(rel_ref[:, 2:3])
    m2 = (m2_3[0] + m2_3[1]) + m2_3[2]
    i2 = gate_c(m2, 2)
    nc2 = i2 * ne + (1.0 - i2) * m2
    o2 = gate_c(nc2, 3)
    z2 = o2 * m2 + (1.0 - o2) * ne
    prob_mid = (_dot_t(z2, rm_ref[...]) / (_znorm(z2) * rnm_ref[...])
                + _take_cols(prob_top, pt_ref, N_TOP, N_MID, BB))

    m1_3 = gather_cell3(rel_ref[:, 1:2])
    m1 = (m1_3[0] + m1_3[1]) + m1_3[2]
    i3 = gate_c(m1, 4)
    nc3 = i3 * ne + (1.0 - i3) * m1
    o3 = gate_c(nc3, 5)
    z3 = o3 * m1 + (1.0 - o3) * ne
    prob_leaf = (_dot_t(z3, rl_ref[...]) / (_znorm(z3) * rnl_ref[...])
                 + _take_cols(prob_mid, pm_ref, N_MID, N_LEAF, BB))

    probs_ref[:, 0:N_LEAF] = _softmax(prob_leaf)
    probs_ref[:, N_LEAF:N_LEAF + N_MID] = _softmax(prob_mid)
    probs_ref[:, N_LEAF + N_MID:NREL] = _softmax(prob_top)
    pred_ref[:, 0:1] = _argmax(prob_leaf, N_LEAF)
    pred_ref[:, 1:2] = _argmax(prob_mid, N_MID)
    pred_ref[:, 2:3] = _argmax(prob_top, N_TOP)


@functools.partial(jax.jit, static_argnames=("interpret",))
def _hme(bag_embedding, relation, head, tail, rel_embs, cell,
         fusion_W, fusion_b, gw_bag, gw_cell, gbias, parent_top, parent_mid,
         interpret=False):
    b = bag_embedding.shape[0]
    grid = (b // BB,)
    full = lambda shape: pl.BlockSpec(shape, lambda i: (0, 0))

    def round_bf16(x):
        u = lax.bitcast_convert_type(x, jnp.uint32)
        u = u + jnp.uint32(0x7FFF) + ((u >> 16) & jnp.uint32(1))
        return lax.bitcast_convert_type(u & jnp.uint32(0xFFFF0000), jnp.float32)

    cell_hi = round_bf16(cell)
    r1 = cell - cell_hi
    cell_md = round_bf16(r1)
    cell_lo = r1 - cell_md

    rn = jnp.maximum(jnp.linalg.norm(rel_embs, axis=-1), 1e-8)
    rn_leaf = rn[:N_LEAF].reshape(1, N_LEAF)
    rn_mid = rn[N_LEAF:N_LEAF + N_MID].reshape(1, N_MID)
    rn_top = rn[N_LEAF + N_MID:].reshape(1, N_TOP)

    pt_lane = (parent_top - (N_LEAF + N_MID)).astype(jnp.int32).reshape(1, N_MID)
    pm_lane = (parent_mid - N_LEAF).astype(jnp.int32).reshape(1, N_LEAF)

    out = pl.pallas_call(
        _hme_block,
        grid=grid,
        in_specs=[
            pl.BlockSpec((BB, SENT), lambda i: (i, 0)),
            pl.BlockSpec((BB, ENT), lambda i: (i, 0)),
            pl.BlockSpec((BB, ENT), lambda i: (i, 0)),
            pl.BlockSpec((BB, 3), lambda i: (i, 0)),
            full((N_LEAF, HIE)),
            full((N_MID, HIE)),
            full((N_TOP, HIE)),
            full((NCELL, HIE)),
            full((NCELL, HIE)),
            full((NCELL, HIE)),
            full((NCELL, HIE)),
            full((SENT, HIE)),
            full((ENT, HIE)),
            full((1, HIE)),
            full((SENT, 6)),
            full((HIE, 6)),
            full((1, 6)),
            full((1, N_LEAF)),
            full((1, N_MID)),
            full((1, N_TOP)),
            full((1, N_MID)),
            full((1, N_LEAF)),
        ],
        out_specs=[
            pl.BlockSpec((BB, NREL), lambda i: (i, 0)),
            pl.BlockSpec((BB, 3), lambda i: (i, 0)),
        ],
        out_shape=[
            jax.ShapeDtypeStruct((b, NREL), jnp.float32),
            jax.ShapeDtypeStruct((b, 3), jnp.int32),
        ],
        interpret=interpret,
    )(bag_embedding, head, tail, relation,
      rel_embs[:N_LEAF], rel_embs[N_LEAF:N_LEAF + N_MID], rel_embs[N_LEAF + N_MID:],
      cell, cell_hi, cell_md, cell_lo,
      fusion_W[:SENT], fusion_W[SENT:], fusion_b.reshape(1, HIE),
      gw_bag, gw_cell, gbias.reshape(1, 6),
      rn_leaf, rn_mid, rn_top, pt_lane, pm_lane)
    return out[0], out[1]


def kernel(bag_embedding, relation, head, tail, rel_embs, cell, fusion_W, fusion_b,
           gi1_w, gi1_b, go1_w, go1_b, gi2_w, gi2_b, go2_w, go2_b,
           gi3_w, gi3_b, go3_w, go3_b, parent_top, parent_mid, interpret=False):
    gws = jnp.stack([gi1_w, go1_w, gi2_w, go2_w, gi3_w, go3_w], axis=-1)
    gbias = jnp.stack([gi1_b, go1_b, gi2_b, go2_b, gi3_b, go3_b])
    return _hme(bag_embedding, relation, head, tail, rel_embs, cell,
                fusion_W, fusion_b, gws[:SENT], gws[SENT:], gbias,
                parent_top, parent_mid, interpret=interpret)
